# baseline (speedup 1.0000x reference)
import functools

import jax
import jax.numpy as jnp
from jax import lax
from jax.experimental import pallas as pl
from jax.experimental.pallas import tpu as pltpu
from jax.experimental.pallas import tpu_sc as plsc

_N, _D = 100000, 128
_B, _S1, _S2 = 1024, 25, 10

_NC, _NS = 2, 16
_NW = _NC * _NS

_PAIRS = _B * _S2 // _NW
_SEEDS = _B // _NW
_CH = 8
_ROWS = _CH * _S1
_NBUF = 4
_NCH_N1 = _PAIRS // _CH
_NCH_SUM = (_PAIRS + _SEEDS) // _CH
_VREGS = _D // 16
_SELF = _PAIRS + _SEEDS
_SELF_H = 176

_MESH = plsc.VectorSubcoreMesh(core_axis_name="c", subcore_axis_name="s",
                               num_cores=_NC, num_subcores=_NS)


def _worker_id():
  return lax.axis_index("c") * _NS + lax.axis_index("s")


def _gather_pieces(rows):
  pieces, o = [], 0
  while o < rows:
    ln = min(128, rows - o)
    pieces.append((o, ln))
    o += ln
  return pieces


def _gather_start(feat, idx_v, idx_off, buf, sem, rows):
  for o, ln in _gather_pieces(rows):
    pltpu.async_copy(feat.at[idx_v.at[pl.ds(idx_off + o, ln)]],
                     buf.at[pl.ds(o, ln)], sem)


def _gather_wait(feat, idx_v, idx_off, buf, sem, rows):
  for o, ln in _gather_pieces(rows):
    pltpu.make_async_copy(feat.at[idx_v.at[pl.ds(idx_off + o, ln)]],
                          buf.at[pl.ds(o, ln)], sem).wait()


_A_SUM = _SELF


def _sc_small_body(feat, n1f, nodes, nnseed, sum_seed, self_n1, self_seed,
                   idx_v, sbuf0, sbuf1, buf0, buf1, stage,
                   ssem0, ssem1, sem0, sem1, osem):
  wid = _worker_id()
  bufs = (buf0, buf1)
  sems = (sem0, sem1)

  def acc_group(buf, stage_base):
    def pair_body(p, carry):
      s = [buf[p * _S1, pl.ds(v * 16, 16)] for v in range(_VREGS)]
      for j in range(1, _S1):
        for v in range(_VREGS):
          s[v] = s[v] + buf[p * _S1 + j, pl.ds(v * 16, 16)]
      for v in range(_VREGS):
        stage[stage_base + p, pl.ds(v * 16, 16)] = s[v]
      return carry
    lax.fori_loop(0, _CH, pair_body, 0)

  i1 = pltpu.async_copy(n1f.at[pl.ds(wid * _PAIRS, _PAIRS)],
                        idx_v.at[pl.ds(0, _PAIRS)], ssem0)
  i2 = pltpu.async_copy(nodes.at[pl.ds(wid * _SEEDS, _SEEDS)],
                        idx_v.at[pl.ds(_PAIRS, _SEEDS)], ssem0)
  i3 = pltpu.async_copy(nnseed.at[pl.ds(wid * _SEEDS * _S1, _SEEDS * _S1)],
                        idx_v.at[pl.ds(_A_SUM, _SEEDS * _S1)], osem)
  i1.wait()
  i2.wait()
  _gather_start(feat, idx_v, 0, sbuf0, ssem0, _SELF_H)
  _gather_start(feat, idx_v, _SELF_H, sbuf1, ssem1, _SELF_H)
  i3.wait()
  for b in range(2):
    _gather_start(feat, idx_v, _A_SUM + b * _ROWS, bufs[b], sems[b], _ROWS)

  writes = []
  for g in range(_SEEDS // _CH):
    b = g % 2
    _gather_wait(feat, idx_v, _A_SUM + g * _ROWS, bufs[b], sems[b], _ROWS)
    acc_group(bufs[b], g * _CH)
    w = pltpu.make_async_copy(
        stage.at[pl.ds(g * _CH, _CH)],
        sum_seed.at[pl.ds(wid * _SEEDS + g * _CH, _CH)], osem)
    w.start()
    writes.append(w)
    if g + 2 < _SEEDS // _CH:
      _gather_start(feat, idx_v, _A_SUM + (g + 2) * _ROWS, bufs[b], sems[b],
                    _ROWS)

  _gather_wait(feat, idx_v, 0, sbuf0, ssem0, _SELF_H)
  w = pltpu.make_async_copy(sbuf0.at[pl.ds(0, _SELF_H)],
                            self_n1.at[pl.ds(wid * _PAIRS, _SELF_H)], osem)
  w.start()
  writes.append(w)
  _gather_wait(feat, idx_v, _SELF_H, sbuf1, ssem1, _SELF_H)
  n1_rest = _PAIRS - _SELF_H
  w = pltpu.make_async_copy(
      sbuf1.at[pl.ds(0, n1_rest)],
      self_n1.at[pl.ds(wid * _PAIRS + _SELF_H, n1_rest)], osem)
  w.start()
  writes.append(w)
  w = pltpu.make_async_copy(sbuf1.at[pl.ds(n1_rest, _SEEDS)],
                            self_seed.at[pl.ds(wid * _SEEDS, _SEEDS)], osem)
  w.start()
  writes.append(w)
  for w in writes:
    w.wait()


def _sc_sum_body(feat, nnn1, sum_n1,
                 idx_v, buf0, buf1, buf2, buf3, stage,
                 sem0, sem1, sem2, sem3, osem):
  wid = _worker_id()
  bufs = (buf0, buf1, buf2, buf3)
  sems = (sem0, sem1, sem2, sem3)

  def acc_group(buf, stage_base):
    def pair_body(p, carry):
      s = [buf[p * _S1, pl.ds(v * 16, 16)] for v in range(_VREGS)]
      for j in range(1, _S1):
        for v in range(_VREGS):
          s[v] = s[v] + buf[p * _S1 + j, pl.ds(v * 16, 16)]
      for v in range(_VREGS):
        stage[stage_base + p, pl.ds(v * 16, 16)] = s[v]
      return carry
    lax.fori_loop(0, _CH, pair_body, 0)

  def n1_write(b, g):
    return pltpu.make_async_copy(
        stage.at[pl.ds(b * _CH, _CH)],
        sum_n1.at[pl.ds(wid * _PAIRS + g * _CH, _CH)], osem)

  head = _NBUF * _ROWS
  rest = _PAIRS * _S1 - head
  base = wid * _PAIRS * _S1
  pltpu.sync_copy(nnn1.at[pl.ds(base, head)], idx_v.at[pl.ds(0, head)])
  for b in range(_NBUF):
    _gather_start(feat, idx_v, b * _ROWS, bufs[b], sems[b], _ROWS)
  pltpu.sync_copy(nnn1.at[pl.ds(base + head, rest)],
                  idx_v.at[pl.ds(head, rest)])

  def loop_body(k, carry):
    g0 = _NBUF * k
    for b in range(_NBUF):
      g = g0 + b
      _gather_wait(feat, idx_v, g * _ROWS, bufs[b], sems[b], _ROWS)

      @pl.when(g >= _NBUF)
      def _():
        n1_write(b, g - _NBUF).wait()

      acc_group(bufs[b], b * _CH)
      n1_write(b, g).start()

      @pl.when(g + _NBUF < _NCH_N1)
      def _():
        _gather_start(feat, idx_v, (g + _NBUF) * _ROWS, bufs[b], sems[b],
                      _ROWS)
    return carry

  lax.fori_loop(0, _NCH_N1 // _NBUF, loop_body, 0)
  for b in range(_NBUF):
    n1_write(b, _NCH_N1 - _NBUF + b).wait()


def _sc_small(feature, n1f, nodes, nnseedf):
  f32 = jnp.float32
  return pl.kernel(
      _sc_small_body,
      out_type=(
          jax.ShapeDtypeStruct((_B, _D), f32),
          jax.ShapeDtypeStruct((_B * _S2, _D), f32),
          jax.ShapeDtypeStruct((_B, _D), f32),
      ),
      mesh=_MESH,
      scratch_types=[
          pltpu.VMEM((_A_SUM + _SEEDS * _S1,), jnp.int32),
          pltpu.VMEM((_SELF_H, _D), f32),
          pltpu.VMEM((_SELF_H, _D), f32),
          pltpu.VMEM((_ROWS, _D), f32),
          pltpu.VMEM((_ROWS, _D), f32),
          pltpu.VMEM((_NBUF * _CH, _D), f32),
          pltpu.SemaphoreType.DMA,
          pltpu.SemaphoreType.DMA,
          pltpu.SemaphoreType.DMA,
          pltpu.SemaphoreType.DMA,
          pltpu.SemaphoreType.DMA,
      ],
  )(feature, n1f, nodes, nnseedf)


def _sc_sums(feature, nnn1f):
  return pl.kernel(
      _sc_sum_body,
      out_type=jax.ShapeDtypeStruct((_B * _S2, _D), jnp.float32),
      mesh=_MESH,
      scratch_types=[
          pltpu.VMEM((_PAIRS * _S1,), jnp.int32),
          pltpu.VMEM((_ROWS, _D), f32 := jnp.float32),
          pltpu.VMEM((_ROWS, _D), f32),
          pltpu.VMEM((_ROWS, _D), f32),
          pltpu.VMEM((_ROWS, _D), f32),
          pltpu.VMEM((_NBUF * _CH, _D), f32),
          pltpu.SemaphoreType.DMA,
          pltpu.SemaphoreType.DMA,
          pltpu.SemaphoreType.DMA,
          pltpu.SemaphoreType.DMA,
          pltpu.SemaphoreType.DMA,
      ],
  )(feature, nnn1f)


_BS = 512


def _l2norm(h):
  ss = jnp.sum(h * h, axis=-1, keepdims=True)
  return h / jnp.maximum(jnp.sqrt(ss), 1e-12)


def _tc_body(ss_ref, sums_ref, sn_ref, sumn_ref, w0_ref, b0_ref, w1_ref,
             b1_ref, o_ref):
  f32 = jnp.float32
  w0 = w0_ref[:]
  w0a, w0b = w0[:_D], w0[_D:]
  b0 = b0_ref[:]
  inv_s1 = 1.0 / _S1

  hs = jnp.dot(ss_ref[:], w0a, preferred_element_type=f32)
  hs += jnp.dot(sums_ref[:] * inv_s1, w0b, preferred_element_type=f32)
  hs = _l2norm(jnp.maximum(hs + b0, 0.0))

  hn = jnp.dot(sn_ref[:], w0a, preferred_element_type=f32)
  hn += jnp.dot(sumn_ref[:] * inv_s1, w0b, preferred_element_type=f32)
  hn = _l2norm(jnp.maximum(hn + b0, 0.0))

  m = jnp.mean(hn.reshape(_BS, _S2, _D), axis=1)
  w1 = w1_ref[:]
  z = jnp.dot(hs, w1[:_D], preferred_element_type=f32)
  z += jnp.dot(m, w1[_D:], preferred_element_type=f32)
  o_ref[:] = _l2norm(jnp.maximum(z + b1_ref[:], 0.0))


def _tc_mlp(self_seed, sum_seed, self_n1, sum_n1, W0, b0, W1, b1):
  grid = (_B // _BS,)
  rep = lambda i: (0, 0)
  return pl.pallas_call(
      _tc_body,
      grid=grid,
      in_specs=[
          pl.BlockSpec((_BS, _D), lambda i: (i, 0)),
          pl.BlockSpec((_BS, _D), lambda i: (i, 0)),
          pl.BlockSpec((_BS * _S2, _D), lambda i: (i, 0)),
          pl.BlockSpec((_BS * _S2, _D), lambda i: (i, 0)),
          pl.BlockSpec((2 * _D, _D), rep),
          pl.BlockSpec((1, _D), rep),
          pl.BlockSpec((2 * _D, _D), rep),
          pl.BlockSpec((1, _D), rep),
      ],
      out_specs=pl.BlockSpec((_BS, _D), lambda i: (i, 0)),
      out_shape=jax.ShapeDtypeStruct((_B, _D), jnp.float32),
  )(self_seed, sum_seed, self_n1, sum_n1, W0, b0, W1, b1)


@jax.jit
def kernel(feature, nodes, n1, nn_seed, nn_n1, W0, b0, W1, b1):
  i32 = jnp.int32
  n1f = n1.astype(i32).reshape(-1)
  nodesf = nodes.astype(i32).reshape(-1)
  nnseedf = nn_seed.astype(i32).reshape(-1)
  nnn1f = nn_n1.astype(i32).reshape(-1)
  sum_seed, self_n1, self_seed = _sc_small(feature, n1f, nodesf, nnseedf)
  sum_n1 = _sc_sums(feature, nnn1f)
  return _tc_mlp(self_seed, sum_seed, self_n1, sum_n1,
                 W0, b0.reshape(1, _D), W1, b1.reshape(1, _D))


# speedup vs baseline: 1.0031x; 1.0031x over previous
import functools

import jax
import jax.numpy as jnp
from jax import lax
from jax.experimental import pallas as pl
from jax.experimental.pallas import tpu as pltpu
from jax.experimental.pallas import tpu_sc as plsc

_N, _D = 100000, 128
_B, _S1, _S2 = 1024, 25, 10

_NC, _NS = 2, 16
_NW = _NC * _NS

_PAIRS = _B * _S2 // _NW
_SEEDS = _B // _NW
_CH = 8
_ROWS = _CH * _S1
_NBUF = 4
_NCH_N1 = _PAIRS // _CH
_NCH_SUM = (_PAIRS + _SEEDS) // _CH
_VREGS = _D // 16
_SELF = _PAIRS + _SEEDS
_SELF_H = 176

_MESH = plsc.VectorSubcoreMesh(core_axis_name="c", subcore_axis_name="s",
                               num_cores=_NC, num_subcores=_NS)


def _worker_id():
  return lax.axis_index("c") * _NS + lax.axis_index("s")


def _gather_pieces(rows):
  pieces, o = [], 0
  while o < rows:
    ln = min(128, rows - o)
    pieces.append((o, ln))
    o += ln
  return pieces


def _gather_start(feat, idx_v, idx_off, buf, sem, rows):
  for o, ln in _gather_pieces(rows):
    pltpu.async_copy(feat.at[idx_v.at[pl.ds(idx_off + o, ln)]],
                     buf.at[pl.ds(o, ln)], sem)


def _gather_wait(feat, idx_v, idx_off, buf, sem, rows):
  for o, ln in _gather_pieces(rows):
    pltpu.make_async_copy(feat.at[idx_v.at[pl.ds(idx_off + o, ln)]],
                          buf.at[pl.ds(o, ln)], sem).wait()


_A_SUM = _SELF


def _sc_small_body(feat, n1f, nodes, nnseed, sum_seed, self_n1, self_seed,
                   idx_v, sbuf0, sbuf1, buf0, buf1, stage,
                   ssem0, ssem1, sem0, sem1, osem):
  wid = _worker_id()
  bufs = (buf0, buf1)
  sems = (sem0, sem1)

  def acc_group(buf, stage_base):
    def pair_body(p, carry):
      s = [buf[p * _S1, pl.ds(v * 16, 16)] for v in range(_VREGS)]
      for j in range(1, _S1):
        for v in range(_VREGS):
          s[v] = s[v] + buf[p * _S1 + j, pl.ds(v * 16, 16)]
      for v in range(_VREGS):
        stage[stage_base + p, pl.ds(v * 16, 16)] = s[v]
      return carry
    lax.fori_loop(0, _CH, pair_body, 0)

  i1 = pltpu.async_copy(n1f.at[pl.ds(wid * _PAIRS, _PAIRS)],
                        idx_v.at[pl.ds(0, _PAIRS)], ssem0)
  i2 = pltpu.async_copy(nodes.at[pl.ds(wid * _SEEDS, _SEEDS)],
                        idx_v.at[pl.ds(_PAIRS, _SEEDS)], ssem0)
  i3 = pltpu.async_copy(nnseed.at[pl.ds(wid * _SEEDS * _S1, _SEEDS * _S1)],
                        idx_v.at[pl.ds(_A_SUM, _SEEDS * _S1)], osem)
  i1.wait()
  i2.wait()
  _gather_start(feat, idx_v, 0, sbuf0, ssem0, _SELF_H)
  _gather_start(feat, idx_v, _SELF_H, sbuf1, ssem1, _SELF_H)
  i3.wait()
  for b in range(2):
    _gather_start(feat, idx_v, _A_SUM + b * _ROWS, bufs[b], sems[b], _ROWS)

  writes = []
  for g in range(_SEEDS // _CH):
    b = g % 2
    _gather_wait(feat, idx_v, _A_SUM + g * _ROWS, bufs[b], sems[b], _ROWS)
    acc_group(bufs[b], g * _CH)
    w = pltpu.make_async_copy(
        stage.at[pl.ds(g * _CH, _CH)],
        sum_seed.at[pl.ds(wid * _SEEDS + g * _CH, _CH)], osem)
    w.start()
    writes.append(w)
    if g + 2 < _SEEDS // _CH:
      _gather_start(feat, idx_v, _A_SUM + (g + 2) * _ROWS, bufs[b], sems[b],
                    _ROWS)

  _gather_wait(feat, idx_v, 0, sbuf0, ssem0, _SELF_H)
  w = pltpu.make_async_copy(sbuf0.at[pl.ds(0, _SELF_H)],
                            self_n1.at[pl.ds(wid * _PAIRS, _SELF_H)], osem)
  w.start()
  writes.append(w)
  _gather_wait(feat, idx_v, _SELF_H, sbuf1, ssem1, _SELF_H)
  n1_rest = _PAIRS - _SELF_H
  w = pltpu.make_async_copy(
      sbuf1.at[pl.ds(0, n1_rest)],
      self_n1.at[pl.ds(wid * _PAIRS + _SELF_H, n1_rest)], osem)
  w.start()
  writes.append(w)
  w = pltpu.make_async_copy(sbuf1.at[pl.ds(n1_rest, _SEEDS)],
                            self_seed.at[pl.ds(wid * _SEEDS, _SEEDS)], osem)
  w.start()
  writes.append(w)
  for w in writes:
    w.wait()


def _sc_sum_body(feat, nnn1, sum_n1,
                 idx_v, buf0, buf1, buf2, buf3, stage,
                 sem0, sem1, sem2, sem3, osem):
  wid = _worker_id()
  bufs = (buf0, buf1, buf2, buf3)
  sems = (sem0, sem1, sem2, sem3)

  def acc_group(buf, stage_base):
    def pair_body(p, carry):
      s = [buf[p * _S1, pl.ds(v * 16, 16)] for v in range(_VREGS)]
      for j in range(1, _S1):
        for v in range(_VREGS):
          s[v] = s[v] + buf[p * _S1 + j, pl.ds(v * 16, 16)]
      for v in range(_VREGS):
        stage[stage_base + p, pl.ds(v * 16, 16)] = s[v]
      return carry
    lax.fori_loop(0, _CH, pair_body, 0)

  def n1_write(b, g):
    return pltpu.make_async_copy(
        stage.at[pl.ds(b * _CH, _CH)],
        sum_n1.at[pl.ds(wid * _PAIRS + g * _CH, _CH)], osem)

  head = _NBUF * _ROWS
  rest = _PAIRS * _S1 - head
  base = wid * _PAIRS * _S1
  pltpu.sync_copy(nnn1.at[pl.ds(base, head)], idx_v.at[pl.ds(0, head)])
  for b in range(_NBUF):
    _gather_start(feat, idx_v, b * _ROWS, bufs[b], sems[b], _ROWS)
  pltpu.sync_copy(nnn1.at[pl.ds(base + head, rest)],
                  idx_v.at[pl.ds(head, rest)])

  def loop_body(k, carry):
    g0 = _NBUF * k
    for b in range(_NBUF):
      g = g0 + b
      _gather_wait(feat, idx_v, g * _ROWS, bufs[b], sems[b], _ROWS)

      @pl.when(g >= _NBUF)
      def _():
        n1_write(b, g - _NBUF).wait()

      acc_group(bufs[b], b * _CH)
      n1_write(b, g).start()

      @pl.when(g + _NBUF < _NCH_N1)
      def _():
        _gather_start(feat, idx_v, (g + _NBUF) * _ROWS, bufs[b], sems[b],
                      _ROWS)
    return carry

  lax.fori_loop(0, _NCH_N1 // _NBUF, loop_body, 0)
  for b in range(_NBUF):
    n1_write(b, _NCH_N1 - _NBUF + b).wait()


def _sc_small(feature, n1f, nodes, nnseedf):
  f32 = jnp.float32
  return pl.kernel(
      _sc_small_body,
      out_type=(
          jax.ShapeDtypeStruct((_B, _D), f32),
          jax.ShapeDtypeStruct((_B * _S2, _D), f32),
          jax.ShapeDtypeStruct((_B, _D), f32),
      ),
      mesh=_MESH,
      scratch_types=[
          pltpu.VMEM((_A_SUM + _SEEDS * _S1,), jnp.int32),
          pltpu.VMEM((_SELF_H, _D), f32),
          pltpu.VMEM((_SELF_H, _D), f32),
          pltpu.VMEM((_ROWS, _D), f32),
          pltpu.VMEM((_ROWS, _D), f32),
          pltpu.VMEM((_NBUF * _CH, _D), f32),
          pltpu.SemaphoreType.DMA,
          pltpu.SemaphoreType.DMA,
          pltpu.SemaphoreType.DMA,
          pltpu.SemaphoreType.DMA,
          pltpu.SemaphoreType.DMA,
      ],
  )(feature, n1f, nodes, nnseedf)


def _sc_sums(feature, nnn1f):
  return pl.kernel(
      _sc_sum_body,
      out_type=jax.ShapeDtypeStruct((_B * _S2, _D), jnp.float32),
      mesh=_MESH,
      scratch_types=[
          pltpu.VMEM((_PAIRS * _S1,), jnp.int32),
          pltpu.VMEM((_ROWS, _D), f32 := jnp.float32),
          pltpu.VMEM((_ROWS, _D), f32),
          pltpu.VMEM((_ROWS, _D), f32),
          pltpu.VMEM((_ROWS, _D), f32),
          pltpu.VMEM((_NBUF * _CH, _D), f32),
          pltpu.SemaphoreType.DMA,
          pltpu.SemaphoreType.DMA,
          pltpu.SemaphoreType.DMA,
          pltpu.SemaphoreType.DMA,
          pltpu.SemaphoreType.DMA,
      ],
  )(feature, nnn1f)


_BS = 512


def _l2norm(h):
  ss = jnp.sum(h * h, axis=-1, keepdims=True)
  return h / jnp.maximum(jnp.sqrt(ss), 1e-12)


def _tc_body(ss_ref, sums_ref, sn_ref, sumn_ref, w0_ref, b0_ref, w1_ref,
             b1_ref, o_ref):
  f32 = jnp.float32
  w0 = w0_ref[:]
  w0a, w0b = w0[:_D], w0[_D:] * (1.0 / _S1)
  b0 = b0_ref[:]

  hs = jnp.dot(ss_ref[:], w0a, preferred_element_type=f32)
  hs += jnp.dot(sums_ref[:], w0b, preferred_element_type=f32)
  hs = _l2norm(jnp.maximum(hs + b0, 0.0))

  hn = jnp.dot(sn_ref[:], w0a, preferred_element_type=f32)
  hn += jnp.dot(sumn_ref[:], w0b, preferred_element_type=f32)
  hn = _l2norm(jnp.maximum(hn + b0, 0.0))

  m = jnp.sum(hn.reshape(_BS, _S2, _D), axis=1)
  w1 = w1_ref[:]
  z = jnp.dot(hs, w1[:_D], preferred_element_type=f32)
  z += jnp.dot(m, w1[_D:] * (1.0 / _S2), preferred_element_type=f32)
  o_ref[:] = _l2norm(jnp.maximum(z + b1_ref[:], 0.0))


def _tc_mlp(self_seed, sum_seed, self_n1, sum_n1, W0, b0, W1, b1):
  grid = (_B // _BS,)
  rep = lambda i: (0, 0)
  return pl.pallas_call(
      _tc_body,
      grid=grid,
      in_specs=[
          pl.BlockSpec((_BS, _D), lambda i: (i, 0)),
          pl.BlockSpec((_BS, _D), lambda i: (i, 0)),
          pl.BlockSpec((_BS * _S2, _D), lambda i: (i, 0)),
          pl.BlockSpec((_BS * _S2, _D), lambda i: (i, 0)),
          pl.BlockSpec((2 * _D, _D), rep),
          pl.BlockSpec((1, _D), rep),
          pl.BlockSpec((2 * _D, _D), rep),
          pl.BlockSpec((1, _D), rep),
      ],
      out_specs=pl.BlockSpec((_BS, _D), lambda i: (i, 0)),
      out_shape=jax.ShapeDtypeStruct((_B, _D), jnp.float32),
  )(self_seed, sum_seed, self_n1, sum_n1, W0, b0, W1, b1)


@jax.jit
def kernel(feature, nodes, n1, nn_seed, nn_n1, W0, b0, W1, b1):
  i32 = jnp.int32
  n1f = n1.astype(i32).reshape(-1)
  nodesf = nodes.astype(i32).reshape(-1)
  nnseedf = nn_seed.astype(i32).reshape(-1)
  nnn1f = nn_n1.astype(i32).reshape(-1)
  sum_seed, self_n1, self_seed = _sc_small(feature, n1f, nodesf, nnseedf)
  sum_n1 = _sc_sums(feature, nnn1f)
  return _tc_mlp(self_seed, sum_seed, self_n1, sum_n1,
                 W0, b0.reshape(1, _D), W1, b1.reshape(1, _D))
